# baseline (speedup 1.0000x reference)
import functools

import jax
import jax.numpy as jnp
from jax.experimental import pallas as pl
from jax.experimental.pallas import tpu as pltpu

_N, _C, _HW = 4, 96, 384 * 384
_J = 16
_HWB = _HW // _J


def _body(mask_ref, pred_ref, tgt_ref, s_ref, a_ref, c_ref):
    n = pl.program_id(0)
    j = pl.program_id(1)

    @pl.when(jnp.logical_and(n == 0, j == 0))
    def _init():
        s_ref[0, 0] = 0.0
        a_ref[0, 0] = 0.0
        c_ref[0, 0] = 0.0

    p = pred_ref[0]
    t = tgt_ref[0]
    m = mask_ref[0]

    ad = jnp.abs(p - t)
    clip = jnp.minimum(ad, 1.0)
    sm = 0.5 * clip * clip + (ad - clip)

    s_ref[0, 0] += jnp.sum(sm * m)
    a_ref[0, 0] += jnp.sum(ad * m)
    c_ref[0, 0] += jnp.sum(m)


@jax.jit
def kernel(pred, target, front_position):
    pred3 = pred.reshape(_N, _C, _HW)
    tgt3 = target.reshape(_N, _C, _HW)
    maskf = front_position.reshape(_N, 1, _HW).astype(jnp.float32)

    scal = jax.ShapeDtypeStruct((1, 1), jnp.float32)
    s_sum, a_sum, m_cnt = pl.pallas_call(
        _body,
        grid=(_N, _J),
        in_specs=[
            pl.BlockSpec((1, 1, _HWB), lambda n, j: (n, 0, j)),
            pl.BlockSpec((1, _C, _HWB), lambda n, j: (n, 0, j)),
            pl.BlockSpec((1, _C, _HWB), lambda n, j: (n, 0, j)),
        ],
        out_specs=[
            pl.BlockSpec(memory_space=pltpu.SMEM),
            pl.BlockSpec(memory_space=pltpu.SMEM),
            pl.BlockSpec(memory_space=pltpu.SMEM),
        ],
        out_shape=[scal, scal, scal],
    )(maskf, pred3, tgt3)

    cnt = m_cnt[0, 0] * _C
    loss = s_sum[0, 0] / cnt
    diff_mean = a_sum[0, 0] / cnt
    return (loss, diff_mean)


# speedup vs baseline: 1.0300x; 1.0300x over previous
import jax
import jax.numpy as jnp
from jax.experimental import pallas as pl
from jax.experimental.pallas import tpu as pltpu

_N, _C, _HW = 4, 96, 384 * 384
_J = 16
_HWB = _HW // _J
_LANEB = 1024
_NJ = _HWB // _LANEB
_NI = _C // 8


def _body(mask_ref, pred_ref, tgt_ref, s_ref, a_ref, c_ref):
    n = pl.program_id(0)
    j = pl.program_id(1)

    @pl.when(jnp.logical_and(n == 0, j == 0))
    def _init():
        s_ref[0, 0] = 0.0
        a_ref[0, 0] = 0.0
        c_ref[0, 0] = 0.0

    acc_s = jnp.zeros((8, _LANEB), jnp.float32)
    acc_a = jnp.zeros((8, _LANEB), jnp.float32)
    acc_m = jnp.zeros((1, _LANEB), jnp.float32)
    for jj in range(_NJ):
        cols = pl.ds(jj * _LANEB, _LANEB)
        m = mask_ref[0, :, cols]
        acc_m = acc_m + m
        mb = jnp.broadcast_to(m, (8, _LANEB))
        for ii in range(_NI):
            rows = pl.ds(ii * 8, 8)
            p = pred_ref[0, rows, cols]
            t = tgt_ref[0, rows, cols]
            ad = jnp.abs(p - t)
            clip = jnp.minimum(ad, 1.0)
            sm = 0.5 * clip * clip + (ad - clip)
            acc_s = acc_s + sm * mb
            acc_a = acc_a + ad * mb

    s_ref[0, 0] += jnp.sum(acc_s)
    a_ref[0, 0] += jnp.sum(acc_a)
    c_ref[0, 0] += jnp.sum(acc_m)


@jax.jit
def kernel(pred, target, front_position):
    pred3 = pred.reshape(_N, _C, _HW)
    tgt3 = target.reshape(_N, _C, _HW)
    maskf = front_position.reshape(_N, 1, _HW).astype(jnp.float32)

    scal = jax.ShapeDtypeStruct((1, 1), jnp.float32)
    s_sum, a_sum, m_cnt = pl.pallas_call(
        _body,
        grid=(_N, _J),
        in_specs=[
            pl.BlockSpec((1, 1, _HWB), lambda n, j: (n, 0, j)),
            pl.BlockSpec((1, _C, _HWB), lambda n, j: (n, 0, j)),
            pl.BlockSpec((1, _C, _HWB), lambda n, j: (n, 0, j)),
        ],
        out_specs=[
            pl.BlockSpec(memory_space=pltpu.SMEM),
            pl.BlockSpec(memory_space=pltpu.SMEM),
            pl.BlockSpec(memory_space=pltpu.SMEM),
        ],
        out_shape=[scal, scal, scal],
    )(maskf, pred3, tgt3)

    cnt = m_cnt[0, 0] * _C
    loss = s_sum[0, 0] / cnt
    diff_mean = a_sum[0, 0] / cnt
    return (loss, diff_mean)


# speedup vs baseline: 4.7408x; 4.6026x over previous
import jax
import jax.numpy as jnp
from jax.experimental import pallas as pl
from jax.experimental.pallas import tpu as pltpu

_N, _C, _H, _W = 4, 96, 384, 384
_J = 16
_HB = _H // _J


def _body(mask_ref, pred_ref, tgt_ref, s_ref, a_ref, c_ref):
    n = pl.program_id(0)
    j = pl.program_id(1)

    @pl.when(jnp.logical_and(n == 0, j == 0))
    def _init():
        s_ref[0, 0] = 0.0
        a_ref[0, 0] = 0.0
        c_ref[0, 0] = 0.0

    m = mask_ref[0, 0]
    acc_s = jnp.zeros((_HB, _W), jnp.float32)
    acc_a = jnp.zeros((_HB, _W), jnp.float32)
    for c in range(_C):
        p = pred_ref[0, c]
        t = tgt_ref[0, c]
        ad = jnp.abs(p - t) * m
        clip = jnp.minimum(ad, 1.0)
        sm = 0.5 * clip * clip + (ad - clip)
        acc_s = acc_s + sm
        acc_a = acc_a + ad

    s_ref[0, 0] += jnp.sum(acc_s)
    a_ref[0, 0] += jnp.sum(acc_a)
    c_ref[0, 0] += jnp.sum(m)


@jax.jit
def kernel(pred, target, front_position):
    maskf = front_position.astype(jnp.float32)

    scal = jax.ShapeDtypeStruct((1, 1), jnp.float32)
    s_sum, a_sum, m_cnt = pl.pallas_call(
        _body,
        grid=(_N, _J),
        in_specs=[
            pl.BlockSpec((1, 1, _HB, _W), lambda n, j: (n, 0, j, 0)),
            pl.BlockSpec((1, _C, _HB, _W), lambda n, j: (n, 0, j, 0)),
            pl.BlockSpec((1, _C, _HB, _W), lambda n, j: (n, 0, j, 0)),
        ],
        out_specs=[
            pl.BlockSpec(memory_space=pltpu.SMEM),
            pl.BlockSpec(memory_space=pltpu.SMEM),
            pl.BlockSpec(memory_space=pltpu.SMEM),
        ],
        out_shape=[scal, scal, scal],
    )(maskf, pred, target)

    cnt = m_cnt[0, 0] * _C
    loss = s_sum[0, 0] / cnt
    diff_mean = a_sum[0, 0] / cnt
    return (loss, diff_mean)


# speedup vs baseline: 5.1259x; 1.0812x over previous
import jax
import jax.numpy as jnp
from jax.experimental import pallas as pl
from jax.experimental.pallas import tpu as pltpu

_N, _C, _H, _W = 4, 96, 384, 384
_J = 8
_HB = _H // _J


def _body(mask_ref, pred_ref, tgt_ref, s_ref, a_ref, c_ref):
    n = pl.program_id(0)
    j = pl.program_id(1)

    @pl.when(jnp.logical_and(n == 0, j == 0))
    def _init():
        s_ref[0, 0] = 0.0
        a_ref[0, 0] = 0.0
        c_ref[0, 0] = 0.0

    m = mask_ref[0, 0]
    acc_s = jnp.zeros((_HB, _W), jnp.float32)
    acc_a = jnp.zeros((_HB, _W), jnp.float32)
    for c in range(_C):
        p = pred_ref[0, c]
        t = tgt_ref[0, c]
        ad = jnp.abs(p - t) * m
        clip = jnp.minimum(ad, 1.0)
        sm = 0.5 * clip * clip + (ad - clip)
        acc_s = acc_s + sm
        acc_a = acc_a + ad

    s_ref[0, 0] += jnp.sum(acc_s)
    a_ref[0, 0] += jnp.sum(acc_a)
    c_ref[0, 0] += jnp.sum(m)


@jax.jit
def kernel(pred, target, front_position):
    maskf = front_position.astype(jnp.float32)

    scal = jax.ShapeDtypeStruct((1, 1), jnp.float32)
    s_sum, a_sum, m_cnt = pl.pallas_call(
        _body,
        grid=(_N, _J),
        in_specs=[
            pl.BlockSpec((1, 1, _HB, _W), lambda n, j: (n, 0, j, 0)),
            pl.BlockSpec((1, _C, _HB, _W), lambda n, j: (n, 0, j, 0)),
            pl.BlockSpec((1, _C, _HB, _W), lambda n, j: (n, 0, j, 0)),
        ],
        out_specs=[
            pl.BlockSpec(memory_space=pltpu.SMEM),
            pl.BlockSpec(memory_space=pltpu.SMEM),
            pl.BlockSpec(memory_space=pltpu.SMEM),
        ],
        out_shape=[scal, scal, scal],
    )(maskf, pred, target)

    cnt = m_cnt[0, 0] * _C
    loss = s_sum[0, 0] / cnt
    diff_mean = a_sum[0, 0] / cnt
    return (loss, diff_mean)


# speedup vs baseline: 5.1721x; 1.0090x over previous
import jax
import jax.numpy as jnp
from jax.experimental import pallas as pl
from jax.experimental.pallas import tpu as pltpu

_N, _C, _H, _W = 4, 96, 384, 384
_J = 6
_HB = _H // _J


def _body(mask_ref, pred_ref, tgt_ref, s_ref, a_ref, c_ref):
    n = pl.program_id(0)
    j = pl.program_id(1)

    @pl.when(jnp.logical_and(n == 0, j == 0))
    def _init():
        s_ref[0, 0] = 0.0
        a_ref[0, 0] = 0.0
        c_ref[0, 0] = 0.0

    m = mask_ref[0, 0]
    acc_s = jnp.zeros((_HB, _W), jnp.float32)
    acc_a = jnp.zeros((_HB, _W), jnp.float32)
    for c in range(_C):
        p = pred_ref[0, c]
        t = tgt_ref[0, c]
        ad = jnp.abs(p - t) * m
        clip = jnp.minimum(ad, 1.0)
        sm = 0.5 * clip * clip + (ad - clip)
        acc_s = acc_s + sm
        acc_a = acc_a + ad

    s_ref[0, 0] += jnp.sum(acc_s)
    a_ref[0, 0] += jnp.sum(acc_a)
    c_ref[0, 0] += jnp.sum(m)


@jax.jit
def kernel(pred, target, front_position):
    scal = jax.ShapeDtypeStruct((1, 1), jnp.float32)
    s_sum, a_sum, m_cnt = pl.pallas_call(
        _body,
        grid=(_N, _J),
        compiler_params=pltpu.CompilerParams(
            vmem_limit_bytes=128 * 1024 * 1024),
        in_specs=[
            pl.BlockSpec((1, 1, _HB, _W), lambda n, j: (n, 0, j, 0)),
            pl.BlockSpec((1, _C, _HB, _W), lambda n, j: (n, 0, j, 0)),
            pl.BlockSpec((1, _C, _HB, _W), lambda n, j: (n, 0, j, 0)),
        ],
        out_specs=[
            pl.BlockSpec(memory_space=pltpu.SMEM),
            pl.BlockSpec(memory_space=pltpu.SMEM),
            pl.BlockSpec(memory_space=pltpu.SMEM),
        ],
        out_shape=[scal, scal, scal],
    )(front_position.astype(jnp.float32), pred, target)

    cnt = m_cnt[0, 0] * _C
    loss = s_sum[0, 0] / cnt
    diff_mean = a_sum[0, 0] / cnt
    return (loss, diff_mean)
